# baseline (speedup 1.0000x reference)
import functools

import jax
import jax.numpy as jnp
from jax import lax
from jax.experimental import pallas as pl
from jax.experimental.pallas import tpu as pltpu
from jax.experimental.pallas import tpu_sc as plsc

N_MODELS = 64
IN_F = 6
OUT_F = 3
HID = 64
N_PTS = 16384
NUM_ANGLE = 8
NUM_HEIGHT = 8
MAX_H = 1.0
MIN_H = -3.0

TILE = 256
GRID = N_PTS // TILE
SUB = 4
RGRID = GRID // SUB
RSUB = 8
RGRID2 = GRID // RSUB
TS = 256
CAP = N_PTS + N_MODELS * TS
CGRID = CAP // TS
MSUB = 8
MGRID = CGRID // MSUB
ROW = 16
XF = 8

NC, NS = 2, 16
NW = NC * NS
PW = N_PTS // NW
KCH = PW // 128


def _routing_bins_row(xr):
    xx = xr[0:1, :]
    yy = xr[1:2, :]
    hh = xr[2:3, :]
    angle = jnp.arctan2(yy, xx)
    af = jnp.clip(jnp.floor((angle + jnp.pi) / (2.0 * jnp.pi) * NUM_ANGLE),
                  0.0, NUM_ANGLE - 1.0)
    hf = jnp.clip(jnp.floor((hh - MIN_H) / (MAX_H - MIN_H) * NUM_HEIGHT),
                  0.0, NUM_HEIGHT - 1.0)
    return af * NUM_HEIGHT + hf


def _routing_bins_lanes(xr):
    return jnp.transpose(_routing_bins_row(xr)).astype(jnp.int32)


def _one_hot(idx):
    eio = lax.broadcasted_iota(jnp.int32, (TILE, N_MODELS), 1)
    return (eio == idx).astype(jnp.float32)


def _counts_kernel(xr_ref, counts_ref):
    t = pl.program_id(0)
    for j in range(SUB):
        idx = _routing_bins_lanes(xr_ref[:, pl.ds(j * TILE, TILE)])
        oh = _one_hot(idx)
        counts_ref[pl.ds(t * SUB + j, 1), :] = (
            jnp.sum(oh, axis=0, keepdims=True).astype(jnp.int32))


def _route_kernel(xr_ref, counts_ref, idx_ref, oh_ref, dest_ref, te_ref,
                  off_s, tp_s, te_s, lt_s):
    t = pl.program_id(0)

    @pl.when(t == 0)
    def _():
        ri = lax.broadcasted_iota(jnp.int32, (TILE, TILE), 0)
        ci = lax.broadcasted_iota(jnp.int32, (TILE, TILE), 1)
        lt_s[...] = (ci < ri).astype(jnp.float32)

        cf = counts_ref[...].astype(jnp.float32)
        r64 = lax.broadcasted_iota(jnp.int32, (GRID, N_MODELS), 0)
        c64 = lax.broadcasted_iota(jnp.int32, (GRID, N_MODELS), 1)
        ltri = (c64 < r64).astype(jnp.float32)
        tp_s[...] = jnp.dot(ltri, cf, preferred_element_type=jnp.float32)
        totals = jnp.sum(cf, axis=0, keepdims=True).astype(jnp.int32)
        caps = ((totals + (TS - 1)) // TS) * TS
        utri = (r64 < c64).astype(jnp.float32)
        off = jnp.dot(caps.astype(jnp.float32), utri,
                      preferred_element_type=jnp.float32)
        off_s[...] = off
        ends = off.astype(jnp.int32) + caps
        ttpos = lax.broadcasted_iota(jnp.int32, (CGRID, N_MODELS), 0) * TS
        tecnt = jnp.sum((ends <= ttpos).astype(jnp.int32), axis=1, keepdims=True)
        te_s[...] = jnp.minimum(tecnt, N_MODELS - 1)

    te_ref[...] = te_s[...]
    lt = lt_s[...]
    idx_rows = []
    dest_rows = []
    for j in range(RSUB):
        idxr = _routing_bins_row(xr_ref[:, pl.ds(j * TILE, TILE)])
        idx_rows.append(idxr)
        idx = jnp.transpose(idxr).astype(jnp.int32)
        oh = _one_hot(idx)
        oh_ref[pl.ds(j * TILE, TILE), :] = oh
        cum = jnp.dot(lt, oh, preferred_element_type=jnp.float32)
        base = off_s[...] + tp_s[pl.ds(t * RSUB + j, 1), :]
        dest = jnp.sum((cum + base) * oh, axis=1, keepdims=True)
        dest_rows.append(jnp.transpose(dest))
    idx_ref[...] = jnp.concatenate(idx_rows, axis=0).astype(jnp.int32)
    dest_ref[...] = jnp.concatenate(dest_rows, axis=0).astype(jnp.int32)


def _sc_scatter(x_hbm, d_hbm, xs_hbm, idx_v, rows_v, sem):
    wid = lax.axis_index("s") * NC + lax.axis_index("c")
    base = wid * PW
    pltpu.sync_copy(d_hbm.at[wid], idx_v)
    pltpu.sync_copy(x_hbm.at[pl.ds(base, PW)], rows_v)
    cps = [pltpu.async_copy(rows_v.at[pl.ds(k * 128, 128)],
                            xs_hbm.at[idx_v.at[k]], sem)
           for k in range(KCH)]
    for cp in cps:
        cp.wait()


def _sc_gather(ys_hbm, d_hbm, out_hbm, idx_v, rows_v, sem):
    wid = lax.axis_index("s") * NC + lax.axis_index("c")
    base = wid * PW
    pltpu.sync_copy(d_hbm.at[wid], idx_v)
    cps = [pltpu.async_copy(ys_hbm.at[idx_v.at[k]],
                            rows_v.at[pl.ds(k * 128, 128)], sem)
           for k in range(KCH)]
    for cp in cps:
        cp.wait()
    pltpu.sync_copy(rows_v, out_hbm.at[pl.ds(base, PW)])


def _sc_mesh():
    return plsc.VectorSubcoreMesh(core_axis_name="c", subcore_axis_name="s",
                                  num_cores=NC, num_subcores=NS)


def _scatter_rows(xp, dest3):
    return pl.kernel(
        _sc_scatter,
        out_type=jax.ShapeDtypeStruct((CAP, ROW), jnp.float32),
        mesh=_sc_mesh(),
        compiler_params=pltpu.CompilerParams(use_tc_tiling_on_sc=False),
        scratch_types=[
            pltpu.VMEM((KCH, 128), jnp.int32),
            pltpu.VMEM((PW, ROW), jnp.float32),
            pltpu.SemaphoreType.DMA,
        ],
    )(xp, dest3)


def _gather_rows(ys, dest3):
    return pl.kernel(
        _sc_gather,
        out_type=jax.ShapeDtypeStruct((N_PTS, ROW), jnp.float32),
        mesh=_sc_mesh(),
        compiler_params=pltpu.CompilerParams(use_tc_tiling_on_sc=False),
        scratch_types=[
            pltpu.VMEM((KCH, 128), jnp.int32),
            pltpu.VMEM((PW, ROW), jnp.float32),
            pltpu.SemaphoreType.DMA,
        ],
    )(ys, dest3)


def _dgT(h, w):
    return lax.dot_general(h, w, (((1,), (1,)), ((), ())),
                           preferred_element_type=jnp.float32)


def _mlp_kernel(te_ref, xs_ref, w0_ref, w1_ref, w2_ref, w3_ref, w4_ref,
                bh_ref, b4_ref, out_ref):
    t = pl.program_id(0)
    for j in range(MSUB):
        e = te_ref[t * MSUB + j]
        h = xs_ref[pl.ds(j * TS, TS), pl.ds(0, XF)]
        h = jnp.maximum(_dgT(h, w0_ref[e]) + bh_ref[e, 0:1, :], 0.0)
        for l, w_ref in enumerate((w1_ref, w2_ref, w3_ref)):
            h = jnp.maximum(_dgT(h, w_ref[e]) + bh_ref[e, l + 1:l + 2, :], 0.0)
        y = _dgT(h, w4_ref[e]) + b4_ref[pl.ds(e, 1), :]
        out_ref[pl.ds(j * TS, TS), pl.ds(0, XF)] = y


@jax.jit
def kernel(x, W0, b0, W1, b1, W2, b2, W3, b3, W4, b4):
    f32 = jnp.float32
    xp = jnp.pad(x, ((0, 0), (0, ROW - IN_F)))
    xr = jnp.pad(x.T, ((0, 2), (0, 0)))

    counts = pl.pallas_call(
        _counts_kernel,
        grid=(RGRID,),
        in_specs=[pl.BlockSpec((8, SUB * TILE), lambda t: (0, t))],
        out_specs=pl.BlockSpec((GRID, N_MODELS), lambda t: (0, 0)),
        out_shape=jax.ShapeDtypeStruct((GRID, N_MODELS), jnp.int32),
        compiler_params=pltpu.CompilerParams(
            dimension_semantics=("arbitrary",)),
    )(xr)

    idx1, oh, dest, te = pl.pallas_call(
        _route_kernel,
        grid=(RGRID2,),
        in_specs=[
            pl.BlockSpec((8, RSUB * TILE), lambda t: (0, t)),
            pl.BlockSpec((GRID, N_MODELS), lambda t: (0, 0)),
        ],
        out_specs=[
            pl.BlockSpec((RSUB, TILE), lambda t: (t, 0)),
            pl.BlockSpec((RSUB * TILE, N_MODELS), lambda t: (t, 0)),
            pl.BlockSpec((RSUB, TILE), lambda t: (t, 0)),
            pl.BlockSpec((CGRID, 1), lambda t: (0, 0)),
        ],
        out_shape=[
            jax.ShapeDtypeStruct((GRID, TILE), jnp.int32),
            jax.ShapeDtypeStruct((N_PTS, N_MODELS), f32),
            jax.ShapeDtypeStruct((GRID, TILE), jnp.int32),
            jax.ShapeDtypeStruct((CGRID, 1), jnp.int32),
        ],
        scratch_shapes=[
            pltpu.VMEM((1, N_MODELS), f32),
            pltpu.VMEM((GRID, N_MODELS), f32),
            pltpu.VMEM((CGRID, 1), jnp.int32),
            pltpu.VMEM((TILE, TILE), f32),
        ],
        compiler_params=pltpu.CompilerParams(
            dimension_semantics=("arbitrary",)),
    )(xr, counts)

    dest3 = dest.reshape(NW, KCH, 128)
    xs = _scatter_rows(xp, dest3)

    w0 = jnp.pad(W0, ((0, 0), (0, 0), (0, XF - IN_F)))
    w4 = jnp.pad(W4, ((0, 0), (0, XF - OUT_F), (0, 0)))
    bh = jnp.stack([b0, b1, b2, b3], axis=1)
    b4p = jnp.pad(b4, ((0, 0), (0, XF - OUT_F)))

    cmap2 = lambda t, te_r: (0, 0)
    cmap3 = lambda t, te_r: (0, 0, 0)
    ys = pl.pallas_call(
        _mlp_kernel,
        grid_spec=pltpu.PrefetchScalarGridSpec(
            num_scalar_prefetch=1,
            grid=(MGRID,),
            in_specs=[
                pl.BlockSpec((MSUB * TS, ROW), lambda t, te_r: (t, 0)),
                pl.BlockSpec((N_MODELS, HID, XF), cmap3),
                pl.BlockSpec((N_MODELS, HID, HID), cmap3),
                pl.BlockSpec((N_MODELS, HID, HID), cmap3),
                pl.BlockSpec((N_MODELS, HID, HID), cmap3),
                pl.BlockSpec((N_MODELS, XF, HID), cmap3),
                pl.BlockSpec((N_MODELS, 4, HID), cmap3),
                pl.BlockSpec((N_MODELS, XF), cmap2),
            ],
            out_specs=pl.BlockSpec((MSUB * TS, ROW), lambda t, te_r: (t, 0)),
        ),
        out_shape=jax.ShapeDtypeStruct((CAP, ROW), f32),
        compiler_params=pltpu.CompilerParams(
            dimension_semantics=("arbitrary",)),
    )(te.reshape(CGRID), xs, w0, W1, W2, W3, w4, bh, b4p)

    outg = _gather_rows(ys, dest3)

    out = outg[:, :OUT_F]
    idx = idx1.reshape(N_PTS)
    return (out, out, idx, oh, oh)


# speedup vs baseline: 1.5214x; 1.5214x over previous
import functools

import jax
import jax.numpy as jnp
from jax import lax
from jax.experimental import pallas as pl
from jax.experimental.pallas import tpu as pltpu
from jax.experimental.pallas import tpu_sc as plsc

N_MODELS = 64
IN_F = 6
OUT_F = 3
HID = 64
N_PTS = 16384
NUM_ANGLE = 8
NUM_HEIGHT = 8
MAX_H = 1.0
MIN_H = -3.0

TILE = 256
GRID = N_PTS // TILE
SUB = 4
RGRID = GRID // SUB
RSUB = 8
RGRID2 = GRID // RSUB
TS = 256
CAP = N_PTS + N_MODELS * TS
CGRID = CAP // TS
MSUB = 8
MGRID = CGRID // MSUB
ROW = 128
XF = 8

NC, NS = 2, 16
NW = NC * NS
PW = N_PTS // NW
KCH = PW // 128


def _routing_bins_row(xr):
    xx = xr[0:1, :]
    yy = xr[1:2, :]
    hh = xr[2:3, :]
    angle = jnp.arctan2(yy, xx)
    af = jnp.clip(jnp.floor((angle + jnp.pi) / (2.0 * jnp.pi) * NUM_ANGLE),
                  0.0, NUM_ANGLE - 1.0)
    hf = jnp.clip(jnp.floor((hh - MIN_H) / (MAX_H - MIN_H) * NUM_HEIGHT),
                  0.0, NUM_HEIGHT - 1.0)
    return af * NUM_HEIGHT + hf


def _routing_bins_lanes(xr):
    return jnp.transpose(_routing_bins_row(xr)).astype(jnp.int32)


def _one_hot(idx):
    eio = lax.broadcasted_iota(jnp.int32, (TILE, N_MODELS), 1)
    return (eio == idx).astype(jnp.float32)


def _counts_kernel(xr_ref, counts_ref):
    t = pl.program_id(0)
    for j in range(SUB):
        idx = _routing_bins_lanes(xr_ref[:, pl.ds(j * TILE, TILE)])
        oh = _one_hot(idx)
        counts_ref[pl.ds(t * SUB + j, 1), :] = (
            jnp.sum(oh, axis=0, keepdims=True).astype(jnp.int32))


def _route_kernel(xr_ref, counts_ref, idx_ref, oh_ref, dest_ref, te_ref,
                  off_s, tp_s, te_s, lt_s):
    t = pl.program_id(0)

    @pl.when(t == 0)
    def _():
        ri = lax.broadcasted_iota(jnp.int32, (TILE, TILE), 0)
        ci = lax.broadcasted_iota(jnp.int32, (TILE, TILE), 1)
        lt_s[...] = (ci < ri).astype(jnp.float32)

        cf = counts_ref[...].astype(jnp.float32)
        r64 = lax.broadcasted_iota(jnp.int32, (GRID, N_MODELS), 0)
        c64 = lax.broadcasted_iota(jnp.int32, (GRID, N_MODELS), 1)
        ltri = (c64 < r64).astype(jnp.float32)
        tp_s[...] = jnp.dot(ltri, cf, preferred_element_type=jnp.float32)
        totals = jnp.sum(cf, axis=0, keepdims=True).astype(jnp.int32)
        caps = ((totals + (TS - 1)) // TS) * TS
        utri = (r64 < c64).astype(jnp.float32)
        off = jnp.dot(caps.astype(jnp.float32), utri,
                      preferred_element_type=jnp.float32)
        off_s[...] = off
        ends = off.astype(jnp.int32) + caps
        ttpos = lax.broadcasted_iota(jnp.int32, (CGRID, N_MODELS), 0) * TS
        tecnt = jnp.sum((ends <= ttpos).astype(jnp.int32), axis=1, keepdims=True)
        te_s[...] = jnp.minimum(tecnt, N_MODELS - 1)

    te_ref[...] = te_s[...]
    lt = lt_s[...]
    idx_rows = []
    dest_rows = []
    for j in range(RSUB):
        idxr = _routing_bins_row(xr_ref[:, pl.ds(j * TILE, TILE)])
        idx_rows.append(idxr)
        idx = jnp.transpose(idxr).astype(jnp.int32)
        oh = _one_hot(idx)
        oh_ref[pl.ds(j * TILE, TILE), :] = oh
        cum = jnp.dot(lt, oh, preferred_element_type=jnp.float32)
        base = off_s[...] + tp_s[pl.ds(t * RSUB + j, 1), :]
        dest = jnp.sum((cum + base) * oh, axis=1, keepdims=True)
        dest_rows.append(jnp.transpose(dest))
    idx_ref[...] = jnp.concatenate(idx_rows, axis=0).astype(jnp.int32)
    dest_ref[...] = jnp.concatenate(dest_rows, axis=0).astype(jnp.int32)


def _sc_scatter(x_hbm, d_hbm, xs_hbm, idx_v, rows_v, sem):
    wid = lax.axis_index("s") * NC + lax.axis_index("c")
    base = wid * PW
    pltpu.sync_copy(d_hbm.at[wid], idx_v)
    pltpu.sync_copy(x_hbm.at[pl.ds(base, PW)], rows_v)
    cps = [pltpu.async_copy(rows_v.at[pl.ds(k * 128, 128)],
                            xs_hbm.at[idx_v.at[k]], sem)
           for k in range(KCH)]
    for cp in cps:
        cp.wait()


def _sc_gather(ys_hbm, d_hbm, out_hbm, idx_v, rows_v, sem):
    wid = lax.axis_index("s") * NC + lax.axis_index("c")
    base = wid * PW
    pltpu.sync_copy(d_hbm.at[wid], idx_v)
    cps = [pltpu.async_copy(ys_hbm.at[idx_v.at[k]],
                            rows_v.at[pl.ds(k * 128, 128)], sem)
           for k in range(KCH)]
    for cp in cps:
        cp.wait()
    pltpu.sync_copy(rows_v, out_hbm.at[pl.ds(base, PW)])


def _sc_mesh():
    return plsc.VectorSubcoreMesh(core_axis_name="c", subcore_axis_name="s",
                                  num_cores=NC, num_subcores=NS)


def _scatter_rows(xp, dest3):
    return pl.kernel(
        _sc_scatter,
        out_type=jax.ShapeDtypeStruct((CAP, ROW), jnp.float32),
        mesh=_sc_mesh(),
        scratch_types=[
            pltpu.VMEM((KCH, 128), jnp.int32),
            pltpu.VMEM((PW, ROW), jnp.float32),
            pltpu.SemaphoreType.DMA,
        ],
    )(xp, dest3)


def _gather_rows(ys, dest3):
    return pl.kernel(
        _sc_gather,
        out_type=jax.ShapeDtypeStruct((N_PTS, ROW), jnp.float32),
        mesh=_sc_mesh(),
        scratch_types=[
            pltpu.VMEM((KCH, 128), jnp.int32),
            pltpu.VMEM((PW, ROW), jnp.float32),
            pltpu.SemaphoreType.DMA,
        ],
    )(ys, dest3)


def _dgT(h, w):
    return lax.dot_general(h, w, (((1,), (1,)), ((), ())),
                           preferred_element_type=jnp.float32)


def _mlp_kernel(te_ref, xs_ref, w0_ref, w1_ref, w2_ref, w3_ref, w4_ref,
                bh_ref, b4_ref, out_ref):
    t = pl.program_id(0)
    es = [te_ref[t * MSUB + j] for j in range(MSUB)]
    hs = [xs_ref[pl.ds(j * TS, TS), pl.ds(0, XF)] for j in range(MSUB)]
    hs = [jnp.maximum(_dgT(h, w0_ref[e]) + bh_ref[e, 0:1, :], 0.0)
          for h, e in zip(hs, es)]
    for l, w_ref in enumerate((w1_ref, w2_ref, w3_ref)):
        hs = [jnp.maximum(_dgT(h, w_ref[e]) + bh_ref[e, l + 1:l + 2, :], 0.0)
              for h, e in zip(hs, es)]
    for j, (h, e) in enumerate(zip(hs, es)):
        y = _dgT(h, w4_ref[e]) + b4_ref[pl.ds(e, 1), :]
        out_ref[pl.ds(j * TS, TS), pl.ds(0, XF)] = y


@jax.jit
def kernel(x, W0, b0, W1, b1, W2, b2, W3, b3, W4, b4):
    f32 = jnp.float32
    xp = jnp.pad(x, ((0, 0), (0, ROW - IN_F)))
    xr = jnp.pad(x.T, ((0, 2), (0, 0)))

    counts = pl.pallas_call(
        _counts_kernel,
        grid=(RGRID,),
        in_specs=[pl.BlockSpec((8, SUB * TILE), lambda t: (0, t))],
        out_specs=pl.BlockSpec((GRID, N_MODELS), lambda t: (0, 0)),
        out_shape=jax.ShapeDtypeStruct((GRID, N_MODELS), jnp.int32),
        compiler_params=pltpu.CompilerParams(
            dimension_semantics=("arbitrary",)),
    )(xr)

    idx1, oh, dest, te = pl.pallas_call(
        _route_kernel,
        grid=(RGRID2,),
        in_specs=[
            pl.BlockSpec((8, RSUB * TILE), lambda t: (0, t)),
            pl.BlockSpec((GRID, N_MODELS), lambda t: (0, 0)),
        ],
        out_specs=[
            pl.BlockSpec((RSUB, TILE), lambda t: (t, 0)),
            pl.BlockSpec((RSUB * TILE, N_MODELS), lambda t: (t, 0)),
            pl.BlockSpec((RSUB, TILE), lambda t: (t, 0)),
            pl.BlockSpec((CGRID, 1), lambda t: (0, 0)),
        ],
        out_shape=[
            jax.ShapeDtypeStruct((GRID, TILE), jnp.int32),
            jax.ShapeDtypeStruct((N_PTS, N_MODELS), f32),
            jax.ShapeDtypeStruct((GRID, TILE), jnp.int32),
            jax.ShapeDtypeStruct((CGRID, 1), jnp.int32),
        ],
        scratch_shapes=[
            pltpu.VMEM((1, N_MODELS), f32),
            pltpu.VMEM((GRID, N_MODELS), f32),
            pltpu.VMEM((CGRID, 1), jnp.int32),
            pltpu.VMEM((TILE, TILE), f32),
        ],
        compiler_params=pltpu.CompilerParams(
            dimension_semantics=("arbitrary",)),
    )(xr, counts)

    dest3 = dest.reshape(NW, KCH, 128)
    xs = _scatter_rows(xp, dest3)

    w0 = jnp.pad(W0, ((0, 0), (0, 0), (0, XF - IN_F)))
    w4 = jnp.pad(W4, ((0, 0), (0, XF - OUT_F), (0, 0)))
    bh = jnp.stack([b0, b1, b2, b3], axis=1)
    b4p = jnp.pad(b4, ((0, 0), (0, XF - OUT_F)))

    cmap2 = lambda t, te_r: (0, 0)
    cmap3 = lambda t, te_r: (0, 0, 0)
    ys = pl.pallas_call(
        _mlp_kernel,
        grid_spec=pltpu.PrefetchScalarGridSpec(
            num_scalar_prefetch=1,
            grid=(MGRID,),
            in_specs=[
                pl.BlockSpec((MSUB * TS, ROW), lambda t, te_r: (t, 0)),
                pl.BlockSpec((N_MODELS, HID, XF), cmap3),
                pl.BlockSpec((N_MODELS, HID, HID), cmap3),
                pl.BlockSpec((N_MODELS, HID, HID), cmap3),
                pl.BlockSpec((N_MODELS, HID, HID), cmap3),
                pl.BlockSpec((N_MODELS, XF, HID), cmap3),
                pl.BlockSpec((N_MODELS, 4, HID), cmap3),
                pl.BlockSpec((N_MODELS, XF), cmap2),
            ],
            out_specs=pl.BlockSpec((MSUB * TS, ROW), lambda t, te_r: (t, 0)),
        ),
        out_shape=jax.ShapeDtypeStruct((CAP, ROW), f32),
        compiler_params=pltpu.CompilerParams(
            dimension_semantics=("arbitrary",)),
    )(te.reshape(CGRID), xs, w0, W1, W2, W3, w4, bh, b4p)

    outg = _gather_rows(ys, dest3)

    out = outg[:, :OUT_F]
    idx = idx1.reshape(N_PTS)
    return (out, out, idx, oh, oh)


# speedup vs baseline: 1.6271x; 1.0695x over previous
import functools

import jax
import jax.numpy as jnp
from jax import lax
from jax.experimental import pallas as pl
from jax.experimental.pallas import tpu as pltpu
from jax.experimental.pallas import tpu_sc as plsc

N_MODELS = 64
IN_F = 6
OUT_F = 3
HID = 64
N_PTS = 16384
NUM_ANGLE = 8
NUM_HEIGHT = 8
MAX_H = 1.0
MIN_H = -3.0

TILE = 256
GRID = N_PTS // TILE
SUB = 4
RGRID = GRID // SUB
RSUB = 8
RGRID2 = GRID // RSUB
TS = 256
CAP = N_PTS + N_MODELS * TS
CGRID = CAP // TS
MSUB = 8
MGRID = CGRID // MSUB
ROW = 128
XF = 8

NC, NS = 2, 16
NW = NC * NS
PW = N_PTS // NW
KCH = PW // 128


def _routing_bins_row(xr):
    xx = xr[0:1, :]
    yy = xr[1:2, :]
    hh = xr[2:3, :]
    angle = jnp.arctan2(yy, xx)
    af = jnp.clip(jnp.floor((angle + jnp.pi) / (2.0 * jnp.pi) * NUM_ANGLE),
                  0.0, NUM_ANGLE - 1.0)
    hf = jnp.clip(jnp.floor((hh - MIN_H) / (MAX_H - MIN_H) * NUM_HEIGHT),
                  0.0, NUM_HEIGHT - 1.0)
    return af * NUM_HEIGHT + hf


def _routing_bins_lanes(xr):
    return jnp.transpose(_routing_bins_row(xr)).astype(jnp.int32)


def _one_hot(idx):
    eio = lax.broadcasted_iota(jnp.int32, (TILE, N_MODELS), 1)
    return (eio == idx).astype(jnp.float32)


def _counts_kernel(xr_ref, counts_ref, xp_ref):
    t = pl.program_id(0)
    for j in range(SUB):
        idx = _routing_bins_lanes(xr_ref[:, pl.ds(j * TILE, TILE)])
        oh = _one_hot(idx)
        counts_ref[pl.ds(t * SUB + j, 1), :] = (
            jnp.sum(oh, axis=0, keepdims=True).astype(jnp.int32))
    xt = jnp.transpose(xr_ref[...])
    xp_ref[...] = jnp.concatenate(
        [xt, jnp.zeros((SUB * TILE, ROW - 8), jnp.float32)], axis=1)


def _route_kernel(xr_ref, counts_ref, idx_ref, oh_ref, dest_ref, te_ref,
                  off_s, tp_s, te_s, lt_s):
    t = pl.program_id(0)

    @pl.when(t == 0)
    def _():
        ri = lax.broadcasted_iota(jnp.int32, (TILE, TILE), 0)
        ci = lax.broadcasted_iota(jnp.int32, (TILE, TILE), 1)
        lt_s[...] = (ci < ri).astype(jnp.float32)

        cf = counts_ref[...].astype(jnp.float32)
        r64 = lax.broadcasted_iota(jnp.int32, (GRID, N_MODELS), 0)
        c64 = lax.broadcasted_iota(jnp.int32, (GRID, N_MODELS), 1)
        ltri = (c64 < r64).astype(jnp.float32)
        tp_s[...] = jnp.dot(ltri, cf, preferred_element_type=jnp.float32)
        totals = jnp.sum(cf, axis=0, keepdims=True).astype(jnp.int32)
        caps = ((totals + (TS - 1)) // TS) * TS
        utri = (r64 < c64).astype(jnp.float32)
        off = jnp.dot(caps.astype(jnp.float32), utri,
                      preferred_element_type=jnp.float32)
        off_s[...] = off
        ends = off.astype(jnp.int32) + caps
        ttpos = lax.broadcasted_iota(jnp.int32, (CGRID, N_MODELS), 0) * TS
        tecnt = jnp.sum((ends <= ttpos).astype(jnp.int32), axis=1, keepdims=True)
        te_s[...] = jnp.minimum(tecnt, N_MODELS - 1)

    te_ref[...] = te_s[...]
    lt = lt_s[...]
    idx_rows = []
    dest_rows = []
    for j in range(RSUB):
        idxr = _routing_bins_row(xr_ref[:, pl.ds(j * TILE, TILE)])
        idx_rows.append(idxr)
        idx = jnp.transpose(idxr).astype(jnp.int32)
        oh = _one_hot(idx)
        oh_ref[pl.ds(j * TILE, TILE), :] = oh
        cum = jnp.dot(lt, oh, preferred_element_type=jnp.float32)
        base = off_s[...] + tp_s[pl.ds(t * RSUB + j, 1), :]
        dest = jnp.sum((cum + base) * oh, axis=1, keepdims=True)
        dest_rows.append(jnp.transpose(dest))
    idx_ref[...] = jnp.concatenate(idx_rows, axis=0).astype(jnp.int32)
    dest_ref[...] = jnp.concatenate(dest_rows, axis=0).astype(jnp.int32)


def _sc_scatter(x_hbm, d_hbm, xs_hbm, idx_v, rows_v, seml, sem):
    wid = lax.axis_index("s") * NC + lax.axis_index("c")
    base = wid * PW
    lds = [pltpu.async_copy(x_hbm.at[pl.ds(base + k * 128, 128)],
                            rows_v.at[pl.ds(k * 128, 128)], seml)
           for k in range(KCH)]
    pltpu.sync_copy(d_hbm.at[wid], idx_v)
    cps = []
    for k in range(KCH):
        lds[k].wait()
        cps.append(pltpu.async_copy(rows_v.at[pl.ds(k * 128, 128)],
                                    xs_hbm.at[idx_v.at[k]], sem))
    for cp in cps:
        cp.wait()


def _sc_gather(ys_hbm, d_hbm, out_hbm, idx_v, rows_v, seml, sem):
    wid = lax.axis_index("s") * NC + lax.axis_index("c")
    base = wid * PW
    pltpu.sync_copy(d_hbm.at[wid], idx_v)
    cps = [pltpu.async_copy(ys_hbm.at[idx_v.at[k]],
                            rows_v.at[pl.ds(k * 128, 128)], sem)
           for k in range(KCH)]
    wrs = []
    for k in range(KCH):
        cps[k].wait()
        wrs.append(pltpu.async_copy(
            rows_v.at[pl.ds(k * 128, 128)],
            out_hbm.at[pl.ds(base + k * 128, 128)], seml))
    for wr in wrs:
        wr.wait()


def _sc_mesh():
    return plsc.VectorSubcoreMesh(core_axis_name="c", subcore_axis_name="s",
                                  num_cores=NC, num_subcores=NS)


def _scatter_rows(xp, dest3):
    return pl.kernel(
        _sc_scatter,
        out_type=jax.ShapeDtypeStruct((CAP, ROW), jnp.float32),
        mesh=_sc_mesh(),
        scratch_types=[
            pltpu.VMEM((KCH, 128), jnp.int32),
            pltpu.VMEM((PW, ROW), jnp.float32),
            pltpu.SemaphoreType.DMA,
            pltpu.SemaphoreType.DMA,
        ],
    )(xp, dest3)


def _gather_rows(ys, dest3):
    return pl.kernel(
        _sc_gather,
        out_type=jax.ShapeDtypeStruct((N_PTS, ROW), jnp.float32),
        mesh=_sc_mesh(),
        scratch_types=[
            pltpu.VMEM((KCH, 128), jnp.int32),
            pltpu.VMEM((PW, ROW), jnp.float32),
            pltpu.SemaphoreType.DMA,
            pltpu.SemaphoreType.DMA,
        ],
    )(ys, dest3)


def _dgT(h, w):
    return lax.dot_general(h, w, (((1,), (1,)), ((), ())),
                           preferred_element_type=jnp.float32)


def _mlp_kernel(te_ref, xs_ref, w0_ref, w1_ref, w2_ref, w3_ref, w4_ref,
                bh_ref, b4_ref, out_ref):
    t = pl.program_id(0)
    es = [te_ref[t * MSUB + j] for j in range(MSUB)]
    hs = [xs_ref[pl.ds(j * TS, TS), pl.ds(0, XF)] for j in range(MSUB)]
    hs = [jnp.maximum(_dgT(h, w0_ref[e]) + bh_ref[e, 0:1, :], 0.0)
          for h, e in zip(hs, es)]
    for l, w_ref in enumerate((w1_ref, w2_ref, w3_ref)):
        hs = [jnp.maximum(_dgT(h, w_ref[e]) + bh_ref[e, l + 1:l + 2, :], 0.0)
              for h, e in zip(hs, es)]
    for j, (h, e) in enumerate(zip(hs, es)):
        y = _dgT(h, w4_ref[e]) + b4_ref[pl.ds(e, 1), :]
        out_ref[pl.ds(j * TS, TS), pl.ds(0, XF)] = y


@jax.jit
def kernel(x, W0, b0, W1, b1, W2, b2, W3, b3, W4, b4):
    f32 = jnp.float32
    xr = jnp.pad(x.T, ((0, 2), (0, 0)))

    counts, xp = pl.pallas_call(
        _counts_kernel,
        grid=(RGRID,),
        in_specs=[pl.BlockSpec((8, SUB * TILE), lambda t: (0, t))],
        out_specs=[
            pl.BlockSpec((GRID, N_MODELS), lambda t: (0, 0)),
            pl.BlockSpec((SUB * TILE, ROW), lambda t: (t, 0)),
        ],
        out_shape=[
            jax.ShapeDtypeStruct((GRID, N_MODELS), jnp.int32),
            jax.ShapeDtypeStruct((N_PTS, ROW), f32),
        ],
        compiler_params=pltpu.CompilerParams(
            dimension_semantics=("arbitrary",)),
    )(xr)

    idx1, oh, dest, te = pl.pallas_call(
        _route_kernel,
        grid=(RGRID2,),
        in_specs=[
            pl.BlockSpec((8, RSUB * TILE), lambda t: (0, t)),
            pl.BlockSpec((GRID, N_MODELS), lambda t: (0, 0)),
        ],
        out_specs=[
            pl.BlockSpec((RSUB, TILE), lambda t: (t, 0)),
            pl.BlockSpec((RSUB * TILE, N_MODELS), lambda t: (t, 0)),
            pl.BlockSpec((RSUB, TILE), lambda t: (t, 0)),
            pl.BlockSpec((CGRID, 1), lambda t: (0, 0)),
        ],
        out_shape=[
            jax.ShapeDtypeStruct((GRID, TILE), jnp.int32),
            jax.ShapeDtypeStruct((N_PTS, N_MODELS), f32),
            jax.ShapeDtypeStruct((GRID, TILE), jnp.int32),
            jax.ShapeDtypeStruct((CGRID, 1), jnp.int32),
        ],
        scratch_shapes=[
            pltpu.VMEM((1, N_MODELS), f32),
            pltpu.VMEM((GRID, N_MODELS), f32),
            pltpu.VMEM((CGRID, 1), jnp.int32),
            pltpu.VMEM((TILE, TILE), f32),
        ],
        compiler_params=pltpu.CompilerParams(
            dimension_semantics=("arbitrary",)),
    )(xr, counts)

    dest3 = dest.reshape(NW, KCH, 128)
    xs = _scatter_rows(xp, dest3)

    w0 = jnp.pad(W0, ((0, 0), (0, 0), (0, XF - IN_F)))
    w4 = jnp.pad(W4, ((0, 0), (0, XF - OUT_F), (0, 0)))
    bh = jnp.stack([b0, b1, b2, b3], axis=1)
    b4p = jnp.pad(b4, ((0, 0), (0, XF - OUT_F)))

    cmap2 = lambda t, te_r: (0, 0)
    cmap3 = lambda t, te_r: (0, 0, 0)
    ys = pl.pallas_call(
        _mlp_kernel,
        grid_spec=pltpu.PrefetchScalarGridSpec(
            num_scalar_prefetch=1,
            grid=(MGRID,),
            in_specs=[
                pl.BlockSpec((MSUB * TS, ROW), lambda t, te_r: (t, 0)),
                pl.BlockSpec((N_MODELS, HID, XF), cmap3),
                pl.BlockSpec((N_MODELS, HID, HID), cmap3),
                pl.BlockSpec((N_MODELS, HID, HID), cmap3),
                pl.BlockSpec((N_MODELS, HID, HID), cmap3),
                pl.BlockSpec((N_MODELS, XF, HID), cmap3),
                pl.BlockSpec((N_MODELS, 4, HID), cmap3),
                pl.BlockSpec((N_MODELS, XF), cmap2),
            ],
            out_specs=pl.BlockSpec((MSUB * TS, ROW), lambda t, te_r: (t, 0)),
        ),
        out_shape=jax.ShapeDtypeStruct((CAP, ROW), f32),
        compiler_params=pltpu.CompilerParams(
            dimension_semantics=("arbitrary",)),
    )(te.reshape(CGRID), xs, w0, W1, W2, W3, w4, bh, b4p)

    outg = _gather_rows(ys, dest3)

    out = outg[:, :OUT_F]
    idx = idx1.reshape(N_PTS)
    return (out, out, idx, oh, oh)
